# baseline (speedup 1.0000x reference)
import functools

import jax
import jax.numpy as jnp
from jax import lax
from jax.experimental import pallas as pl
from jax.experimental.pallas import tpu as pltpu
from jax.experimental.pallas import tpu_sc as plsc

_NC = 2
_NS = 16
_NW = _NC * _NS
_L = 16


def _tc_body(x_ref, p_ref, o_ref):
    o_ref[...] = x_ref[...] + p_ref[...]


def _tc_add(x, pos_embedding, sblk=512):
    B, S, D = x.shape
    grid = (S // sblk, B)
    return pl.pallas_call(
        _tc_body,
        grid=grid,
        in_specs=[
            pl.BlockSpec((1, sblk, D), lambda i, b: (b, i, 0)),
            pl.BlockSpec((sblk, D), lambda i, b: (i, 0)),
        ],
        out_specs=pl.BlockSpec((1, sblk, D), lambda i, b: (b, i, 0)),
        out_shape=jax.ShapeDtypeStruct((B, S, D), x.dtype),
    )(x, pos_embedding)


def _sc_add(x, pos_embedding, r_chunk=8):
    B, S, D = x.shape
    xf = x.reshape(B * S, D)
    rows_per_w = S // _NW
    R = r_chunk
    n_chunks = rows_per_w // R
    vregs_per_row = D // _L

    mesh = plsc.VectorSubcoreMesh(core_axis_name="c", subcore_axis_name="s")

    @functools.partial(
        pl.kernel,
        out_type=jax.ShapeDtypeStruct((B * S, D), jnp.float32),
        mesh=mesh,
        scratch_types=[
            pltpu.VMEM((R, D), jnp.float32),
            pltpu.VMEM((R, D), jnp.float32),
        ],
    )
    def k(x_hbm, pos_hbm, out_hbm, pos_v, xbuf_v):
        wid = lax.axis_index("s") * _NC + lax.axis_index("c")
        s_base = wid * rows_per_w

        def chunk_body(c, carry):
            s0 = s_base + c * R
            pltpu.sync_copy(pos_hbm.at[pl.ds(s0, R)], pos_v)

            def batch_body(b, carry2):
                row0 = b * S + s0
                pltpu.sync_copy(x_hbm.at[pl.ds(row0, R)], xbuf_v)

                def vec_body(j, carry3):
                    col = j * _L
                    for r in range(R):
                        plsc.addupdate(
                            xbuf_v.at[r, pl.ds(col, _L)],
                            pos_v[r, pl.ds(col, _L)],
                        )
                    return carry3

                lax.fori_loop(0, vregs_per_row, vec_body, 0, unroll=4)
                pltpu.sync_copy(xbuf_v, out_hbm.at[pl.ds(row0, R)])
                return carry2

            return lax.fori_loop(0, B, batch_body, carry)

        lax.fori_loop(0, n_chunks, chunk_body, 0)

    return k(xf, pos_embedding).reshape(B, S, D)


def kernel(x, pos_embedding):
    return _sc_add(x, pos_embedding)


# speedup vs baseline: 1.0333x; 1.0333x over previous
import functools

import jax
import jax.numpy as jnp
from jax import lax
from jax.experimental import pallas as pl
from jax.experimental.pallas import tpu as pltpu
from jax.experimental.pallas import tpu_sc as plsc

_NC = 2
_NS = 16
_NW = _NC * _NS
_L = 16


def _tc_body(x_ref, p_ref, o_ref):
    o_ref[...] = x_ref[...] + p_ref[...]


def _tc_add(x, pos_embedding, sblk=512):
    B, S, D = x.shape
    grid = (S // sblk, B)
    return pl.pallas_call(
        _tc_body,
        grid=grid,
        in_specs=[
            pl.BlockSpec((1, sblk, D), lambda i, b: (b, i, 0)),
            pl.BlockSpec((sblk, D), lambda i, b: (i, 0)),
        ],
        out_specs=pl.BlockSpec((1, sblk, D), lambda i, b: (b, i, 0)),
        out_shape=jax.ShapeDtypeStruct((B, S, D), x.dtype),
    )(x, pos_embedding)


def _sc_add(x, pos_embedding):
    B, S, D = x.shape
    xf = x.reshape(B * S, D)
    rows_per_w = S // _NW
    R = 2
    n_chunks = rows_per_w // R
    n_steps = n_chunks * B
    n_iters = n_steps // (2 * B)

    mesh = plsc.VectorSubcoreMesh(core_axis_name="c", subcore_axis_name="s")

    @functools.partial(
        pl.kernel,
        out_type=jax.ShapeDtypeStruct((B * S, D), jnp.float32),
        mesh=mesh,
        scratch_types=(
            [pltpu.VMEM((R, D), jnp.float32)] * 4
            + [pltpu.VMEM((R, D), jnp.float32)] * 4
            + [pltpu.VMEM((R, D), jnp.float32)] * 2
            + [pltpu.SemaphoreType.DMA] * 10
        ),
    )
    def k(x_hbm, pos_hbm, out_hbm, *scr):
        inb = scr[0:4]
        outb = scr[4:8]
        posb = scr[8:10]
        in_sem = scr[10:14]
        out_sem = scr[14:18]
        pos_sem = scr[18:20]

        wid = lax.axis_index("s") * _NC + lax.axis_index("c")
        s_base = wid * rows_per_w

        def in_copy(c, b):
            row0 = b * S + s_base + c * R
            return pltpu.make_async_copy(
                x_hbm.at[pl.ds(row0, R)], inb[b], in_sem[b])

        def out_copy(c, b):
            row0 = b * S + s_base + c * R
            return pltpu.make_async_copy(
                outb[b], out_hbm.at[pl.ds(row0, R)], out_sem[b])

        def pos_copy(c, pc):
            return pltpu.make_async_copy(
                pos_hbm.at[pl.ds(s_base + c * R, R)], posb[pc], pos_sem[pc])

        pos_copy(0, 0).start()
        pos_copy(1, 1).start()
        for b in range(B):
            in_copy(0, b).start()

        def period(g, carry):
            for k_slot in range(2 * B):
                pc = k_slot // B
                b = k_slot % B
                c = 2 * g + pc
                if b == 0:
                    pos_copy(c, pc).wait()
                in_copy(c, b).wait()
                if k_slot >= B:
                    out_copy(c - 1, b).wait()
                else:
                    @pl.when(g > 0)
                    def _():
                        out_copy(c - 1, b).wait()

                def vec_body(j, carry3):
                    col = j * _L
                    for r in range(R):
                        outb[b][r, pl.ds(col, _L)] = (
                            inb[b][r, pl.ds(col, _L)]
                            + posb[pc][r, pl.ds(col, _L)]
                        )
                    return carry3

                lax.fori_loop(0, D // _L, vec_body, 0, unroll=8)
                out_copy(c, b).start()

                if b == B - 1:
                    @pl.when(c + 2 < n_chunks)
                    def _():
                        pos_copy(c + 2, pc).start()

                if k_slot < B:
                    in_copy(c + 1, b).start()
                else:
                    @pl.when(g < n_iters - 1)
                    def _():
                        in_copy(c + 1, b).start()
            return carry

        lax.fori_loop(0, n_iters, period, 0)

        for b in range(B):
            out_copy(n_chunks - 1, b).wait()

    return k(xf, pos_embedding).reshape(B, S, D)


def kernel(x, pos_embedding):
    return _sc_add(x, pos_embedding)


# speedup vs baseline: 3.1459x; 3.0445x over previous
import functools

import jax
import jax.numpy as jnp
from jax import lax
from jax.experimental import pallas as pl
from jax.experimental.pallas import tpu as pltpu
from jax.experimental.pallas import tpu_sc as plsc

_NC = 2
_NS = 16
_NW = _NC * _NS
_L = 16


def _tc_body(x_ref, p_ref, o_ref):
    o_ref[...] = x_ref[...] + p_ref[...]


def _tc_add(x, pos_embedding, sblk=512):
    B, S, D = x.shape
    grid = (S // sblk, B)
    return pl.pallas_call(
        _tc_body,
        grid=grid,
        in_specs=[
            pl.BlockSpec((1, sblk, D), lambda i, b: (b, i, 0)),
            pl.BlockSpec((sblk, D), lambda i, b: (i, 0)),
        ],
        out_specs=pl.BlockSpec((1, sblk, D), lambda i, b: (b, i, 0)),
        out_shape=jax.ShapeDtypeStruct((B, S, D), x.dtype),
    )(x, pos_embedding)


def _sc_add(x, pos_embedding):
    B, S, D = x.shape
    xf = x.reshape(B * S, D)
    rows_per_w = S // _NW
    R = 2
    n_chunks = rows_per_w // R
    n_steps = n_chunks * B
    n_iters = n_steps // (2 * B)

    mesh = plsc.VectorSubcoreMesh(core_axis_name="c", subcore_axis_name="s")

    @functools.partial(
        pl.kernel,
        out_type=jax.ShapeDtypeStruct((B * S, D), jnp.float32),
        mesh=mesh,
        scratch_types=(
            [pltpu.VMEM((R, D), jnp.float32)] * 4
            + [pltpu.VMEM((R, D), jnp.float32)] * 4
            + [pltpu.VMEM((R, D), jnp.float32)] * 2
            + [pltpu.SemaphoreType.DMA] * 10
        ),
    )
    def k(x_hbm, pos_hbm, out_hbm, *scr):
        inb = scr[0:4]
        outb = scr[4:8]
        posb = scr[8:10]
        in_sem = scr[10:14]
        out_sem = scr[14:18]
        pos_sem = scr[18:20]

        wid = lax.axis_index("s") * _NC + lax.axis_index("c")
        s_base = wid * rows_per_w

        def in_copy(c, b):
            row0 = b * S + s_base + c * R
            return pltpu.make_async_copy(
                x_hbm.at[pl.ds(row0, R)], inb[b], in_sem[b])

        def out_copy(c, b):
            row0 = b * S + s_base + c * R
            return pltpu.make_async_copy(
                outb[b], out_hbm.at[pl.ds(row0, R)], out_sem[b])

        def pos_copy(c, pc):
            return pltpu.make_async_copy(
                pos_hbm.at[pl.ds(s_base + c * R, R)], posb[pc], pos_sem[pc])

        pos_copy(0, 0).start()
        pos_copy(1, 1).start()
        for b in range(B):
            in_copy(0, b).start()

        def period(g, carry):
            for k_slot in range(2 * B):
                pc = k_slot // B
                b = k_slot % B
                c = 2 * g + pc
                if b == 0:
                    pos_copy(c, pc).wait()
                in_copy(c, b).wait()
                if k_slot >= B:
                    out_copy(c - 1, b).wait()
                else:
                    @pl.when(g > 0)
                    def _():
                        out_copy(c - 1, b).wait()

                @plsc.parallel_loop(0, D // _L, unroll=8)
                def _(j):
                    col = j * _L
                    for r in range(R):
                        outb[b][r, pl.ds(col, _L)] = (
                            inb[b][r, pl.ds(col, _L)]
                            + posb[pc][r, pl.ds(col, _L)]
                        )
                out_copy(c, b).start()

                if b == B - 1:
                    @pl.when(c + 2 < n_chunks)
                    def _():
                        pos_copy(c + 2, pc).start()

                if k_slot < B:
                    in_copy(c + 1, b).start()
                else:
                    @pl.when(g < n_iters - 1)
                    def _():
                        in_copy(c + 1, b).start()
            return carry

        lax.fori_loop(0, n_iters, period, 0)

        for b in range(B):
            out_copy(n_chunks - 1, b).wait()

    return k(xf, pos_embedding).reshape(B, S, D)


def kernel(x, pos_embedding):
    return _sc_add(x, pos_embedding)


# speedup vs baseline: 3.1489x; 1.0010x over previous
import functools

import jax
import jax.numpy as jnp
from jax import lax
from jax.experimental import pallas as pl
from jax.experimental.pallas import tpu as pltpu
from jax.experimental.pallas import tpu_sc as plsc

_NC = 2
_NS = 16
_NW = _NC * _NS
_L = 16


def _tc_body(x_ref, p_ref, o_ref):
    o_ref[...] = x_ref[...] + p_ref[...]


def _tc_add(x, pos_embedding, sblk=512):
    B, S, D = x.shape
    grid = (S // sblk, B)
    return pl.pallas_call(
        _tc_body,
        grid=grid,
        in_specs=[
            pl.BlockSpec((1, sblk, D), lambda i, b: (b, i, 0)),
            pl.BlockSpec((sblk, D), lambda i, b: (i, 0)),
        ],
        out_specs=pl.BlockSpec((1, sblk, D), lambda i, b: (b, i, 0)),
        out_shape=jax.ShapeDtypeStruct((B, S, D), x.dtype),
    )(x, pos_embedding)


def _sc_add(x, pos_embedding):
    B, S, D = x.shape
    xf = x.reshape(B * S, D)
    rows_per_w = S // _NW
    R = 2
    n_chunks = rows_per_w // R
    n_steps = n_chunks * B
    n_iters = n_steps // (2 * B)

    mesh = plsc.VectorSubcoreMesh(core_axis_name="c", subcore_axis_name="s")

    @functools.partial(
        pl.kernel,
        out_type=jax.ShapeDtypeStruct((B * S, D), jnp.float32),
        mesh=mesh,
        scratch_types=(
            [pltpu.VMEM((R, D), jnp.float32)] * 8
            + [pltpu.VMEM((R, D), jnp.float32)] * 2
            + [pltpu.SemaphoreType.DMA] * 18
        ),
    )
    def k(x_hbm, pos_hbm, out_hbm, *scr):
        iob = scr[0:8]
        posb = scr[8:10]
        in_sem = scr[10:18]
        out_sem = scr[18:26]
        pos_sem = scr[26:28]

        wid = lax.axis_index("s") * _NC + lax.axis_index("c")
        s_base = wid * rows_per_w

        def in_copy(c, b, slot):
            row0 = b * S + s_base + c * R
            return pltpu.make_async_copy(
                x_hbm.at[pl.ds(row0, R)], iob[slot], in_sem[slot])

        def out_copy(c, b, slot):
            row0 = b * S + s_base + c * R
            return pltpu.make_async_copy(
                iob[slot], out_hbm.at[pl.ds(row0, R)], out_sem[slot])

        def pos_copy(c, pc):
            return pltpu.make_async_copy(
                pos_hbm.at[pl.ds(s_base + c * R, R)], posb[pc], pos_sem[pc])

        pos_copy(0, 0).start()
        pos_copy(1, 1).start()
        for b in range(B):
            in_copy(0, b, b).start()

        def period(g, carry):
            for k_slot in range(2 * B):
                pc = k_slot // B
                b = k_slot % B
                c = 2 * g + pc
                if b == 0:
                    pos_copy(c, pc).wait()
                in_copy(c, b, k_slot).wait()

                @plsc.parallel_loop(0, D // _L, unroll=8)
                def _(j):
                    col = j * _L
                    for r in range(R):
                        plsc.addupdate(
                            iob[k_slot].at[r, pl.ds(col, _L)],
                            posb[pc][r, pl.ds(col, _L)],
                        )
                out_copy(c, b, k_slot).start()

                if b == B - 1:
                    @pl.when(c + 2 < n_chunks)
                    def _():
                        pos_copy(c + 2, pc).start()

                if k_slot < B:
                    @pl.when(g > 0)
                    def _():
                        out_copy(2 * g - 1, b, k_slot + B).wait()
                    in_copy(2 * g + 1, b, k_slot + B).start()
                else:
                    out_copy(2 * g, b, k_slot - B).wait()

                    @pl.when(g < n_iters - 1)
                    def _():
                        in_copy(2 * g + 2, b, k_slot - B).start()
            return carry

        lax.fori_loop(0, n_iters, period, 0)

        for b in range(B):
            out_copy(n_chunks - 1, b, b + B).wait()

    return k(xf, pos_embedding).reshape(B, S, D)


def kernel(x, pos_embedding):
    return _sc_add(x, pos_embedding)
